# baseline (speedup 1.0000x reference)
import functools

import jax
import jax.numpy as jnp
import numpy as np
from jax import lax
from jax.experimental import pallas as pl
from jax.experimental.pallas import tpu as pltpu
from jax.experimental.pallas import tpu_sc as plsc

N = 10000
E = 160000
D = 128
ED = 16
T = 4
HEADS = 32

NC = 2
NS = 16
NW = NC * NS
CH = 128
CHUNKS = 40
EPW = CH * CHUNKS
CHB = 64
CHUNKS_B = EPW // CHB
KB0 = 56
KB1 = 104
KBMAX = max(KB0, KB1)
E_PAD = NW * EPW
N_PAD = 10240
ROWS_PER_SUB = N_PAD // NS
NEG = -1.0e30


def _silu(x):
    return x * jax.nn.sigmoid(x)


def _tc1_body(h_ref, w_ref, o_ref):
    acc = h_ref[:, 0, :] * w_ref[0]
    for c in range(1, T):
        acc = acc + h_ref[:, c, :] * w_ref[c]
    o_ref[...] = acc


def _node_emb(h, conv_w):
    bn = 1000
    return pl.pallas_call(
        _tc1_body,
        grid=(N // bn,),
        in_specs=[
            pl.BlockSpec((bn, T, D), lambda i: (i, 0, 0)),
            pl.BlockSpec(memory_space=pltpu.SMEM),
        ],
        out_specs=pl.BlockSpec((bn, D), lambda i: (i, 0)),
        out_shape=jax.ShapeDtypeStruct((N, D), jnp.float32),
    )(h, conv_w)


def _sca_body(ne_hbm, rows3_hbm, cols3_hbm, gr_hbm, gc_hbm,
              ridx_v, cidx_v, br0, bc0, br1, bc1, sg0, sg1, ss0, ss1):
    c = lax.axis_index("c")
    s = lax.axis_index("s")
    wid = c * NS + s
    base0 = wid * EPW
    pltpu.sync_copy(rows3_hbm.at[wid], ridx_v)
    pltpu.sync_copy(cols3_hbm.at[wid], cidx_v)

    def issue_g(j, br, bc, sg):
        pltpu.async_copy(ne_hbm.at[ridx_v.at[j]], br, sg)
        pltpu.async_copy(ne_hbm.at[cidx_v.at[j]], bc, sg)

    def wait_g(br, bc, sg):
        pltpu.make_async_copy(ne_hbm.at[pl.ds(0, CH)], br, sg).wait()
        pltpu.make_async_copy(ne_hbm.at[pl.ds(0, CH)], bc, sg).wait()

    def wait_s(br, bc, ss):
        pltpu.make_async_copy(br, gr_hbm.at[pl.ds(0, CH)], ss).wait()
        pltpu.make_async_copy(bc, gc_hbm.at[pl.ds(0, CH)], ss).wait()

    issue_g(0, br0, bc0, sg0)
    issue_g(1, br1, bc1, sg1)

    @pl.loop(0, CHUNKS, step=2)
    def _(j0):
        base = base0 + j0 * CH
        wait_g(br0, bc0, sg0)
        pltpu.async_copy(br0, gr_hbm.at[pl.ds(base, CH)], ss0)
        pltpu.async_copy(bc0, gc_hbm.at[pl.ds(base, CH)], ss0)
        wait_g(br1, bc1, sg1)
        pltpu.async_copy(br1, gr_hbm.at[pl.ds(base + CH, CH)], ss1)
        pltpu.async_copy(bc1, gc_hbm.at[pl.ds(base + CH, CH)], ss1)
        wait_s(br0, bc0, ss0)

        @pl.when(j0 + 2 < CHUNKS)
        def _():
            issue_g(j0 + 2, br0, bc0, sg0)

        wait_s(br1, bc1, ss1)

        @pl.when(j0 + 3 < CHUNKS)
        def _():
            issue_g(j0 + 3, br1, bc1, sg1)


def _gather_pair(node_emb, rows3, cols3):
    mesh = plsc.VectorSubcoreMesh(core_axis_name="c", subcore_axis_name="s")
    f = pl.kernel(
        _sca_body,
        out_type=[
            jax.ShapeDtypeStruct((E_PAD, D), jnp.float32),
            jax.ShapeDtypeStruct((E_PAD, D), jnp.float32),
        ],
        mesh=mesh,
        scratch_types=[
            pltpu.VMEM((CHUNKS, CH), jnp.int32),
            pltpu.VMEM((CHUNKS, CH), jnp.int32),
            pltpu.VMEM((CH, D), jnp.float32),
            pltpu.VMEM((CH, D), jnp.float32),
            pltpu.VMEM((CH, D), jnp.float32),
            pltpu.VMEM((CH, D), jnp.float32),
            pltpu.SemaphoreType.DMA,
            pltpu.SemaphoreType.DMA,
            pltpu.SemaphoreType.DMA,
            pltpu.SemaphoreType.DMA,
        ],
    )
    return f(node_emb, rows3, cols3)


def _tc2_body(gr, gc, ea, mf, w1a, w1b, w1c, b1, w2, b2, w3, b3, wec, bec,
              rm, mij_ref, e_ref):
    bf = jnp.bfloat16
    f32 = jnp.float32
    dot = functools.partial(jnp.dot, preferred_element_type=f32)
    y = (dot(gr[...].astype(bf), w1a[...]) + dot(gc[...].astype(bf), w1b[...])
         + dot(ea[...], w1c[...]) + b1[...])
    y = _silu(y)
    y = dot(y.astype(bf), w2[...]) + b2[...]
    y = _silu(y)
    m = dot(y.astype(bf), w3[...]) + b3[...]
    mij_ref[...] = m
    e4 = dot(m.astype(bf), wec[...]) + bec[...]
    mrep = lax.dot_general(mf[...], rm[...], (((0,), (0,)), ((), ())),
                           preferred_element_type=f32)
    e4 = jnp.where(mrep > 0.5, e4, NEG)
    for t in range(T):
        e_ref[t, :, :] = e4[:, t * D:(t + 1) * D]


def _edge_mlp(gr, gc, ea_p, mf_p, wd):
    be = 1024
    grid = (E_PAD // be,)
    const = lambda *shape: pl.BlockSpec(shape, lambda i: tuple(0 for _ in shape))
    return pl.pallas_call(
        _tc2_body,
        grid=grid,
        in_specs=[
            pl.BlockSpec((be, D), lambda i: (i, 0)),
            pl.BlockSpec((be, D), lambda i: (i, 0)),
            pl.BlockSpec((be, ED), lambda i: (i, 0)),
            pl.BlockSpec((T, be), lambda i: (0, i)),
            const(D, 3 * D),
            const(D, 3 * D),
            const(ED, 3 * D),
            const(1, 3 * D),
            const(3 * D, 2 * D),
            const(1, 2 * D),
            const(2 * D, D),
            const(1, D),
            const(D, T * D),
            const(1, T * D),
            const(T, T * D),
        ],
        out_specs=[
            pl.BlockSpec((be, D), lambda i: (i, 0)),
            pl.BlockSpec((T, be, D), lambda i: (0, i, 0)),
        ],
        out_shape=[
            jax.ShapeDtypeStruct((E_PAD, D), jnp.float32),
            jax.ShapeDtypeStruct((T, E_PAD, D), jnp.float32),
        ],
    )(gr, gc, ea_p, mf_p, *wd)


def _scb_body(rows2_hbm, cols2_hbm, h0, h1, h2, h3, e_hbm, z_hbm, out_hbm,
              accum, ridx_v, cidx0, cidx1, eb0, xb0, eb1, xb1,
              se0, se1, sg0, sg1, ss0, ss1):
    c = lax.axis_index("c")
    s = lax.axis_index("s")
    hts = (h0, h1, h2, h3)

    for core in range(NC):
        K = (KB0, KB1)[core]
        cbase = s * KB0 if core == 0 else NS * KB0 + s * KB1

        @pl.when(c == core)
        def _(K=K, cbase=cbase):
            pltpu.sync_copy(rows2_hbm.at[pl.ds(cbase, K)],
                            ridx_v.at[pl.ds(0, K)])
            for t in range(T):
                ht = hts[t]

                def issue_pair(j, eb, xb, cidx, se, sg):
                    ch = cbase + j
                    pltpu.async_copy(e_hbm.at[t, pl.ds(ch * CHB, CHB)], eb, se)
                    pltpu.async_copy(cols2_hbm.at[ch, 0], cidx, se)
                    pltpu.async_copy(ht.at[ridx_v.at[j]], xb, sg)

                def half(j, eb, xb, cidx, se, sg, ss):
                    pltpu.make_async_copy(e_hbm.at[0, pl.ds(0, CHB)],
                                          eb, se).wait()
                    pltpu.make_async_copy(cols2_hbm.at[0, 0], cidx, se).wait()
                    pltpu.make_async_copy(ht.at[pl.ds(0, CHB)], xb, sg).wait()

                    @plsc.parallel_loop(0, CHB, unroll=2)
                    def _(r):
                        for q in range(D // 16):
                            sl = pl.ds(q * 16, 16)
                            eb[r, sl] = jnp.maximum(eb[r, sl] + xb[r, sl], 0.0)

                    pltpu.async_copy(eb, accum.at[cidx], ss, add=True)

                pltpu.sync_copy(z_hbm.at[pl.ds(0, ROWS_PER_SUB)],
                                accum.at[pl.ds(s * ROWS_PER_SUB, ROWS_PER_SUB)])
                plsc.subcore_barrier()

                issue_pair(0, eb0, xb0, cidx0, se0, sg0)
                issue_pair(1, eb1, xb1, cidx1, se1, sg1)

                @pl.loop(0, K, step=2)
                def _(j0):
                    half(j0, eb0, xb0, cidx0, se0, sg0, ss0)
                    half(j0 + 1, eb1, xb1, cidx1, se1, sg1, ss1)
                    pltpu.make_async_copy(eb0, accum.at[cidx0], ss0).wait()

                    @pl.when(j0 + 2 < K)
                    def _():
                        issue_pair(j0 + 2, eb0, xb0, cidx0, se0, sg0)

                    pltpu.make_async_copy(eb1, accum.at[cidx1], ss1).wait()

                    @pl.when(j0 + 3 < K)
                    def _():
                        issue_pair(j0 + 3, eb1, xb1, cidx1, se1, sg1)

                plsc.subcore_barrier()
                pltpu.sync_copy(
                    accum.at[pl.ds(s * ROWS_PER_SUB, ROWS_PER_SUB)],
                    out_hbm.at[t, c, pl.ds(s * ROWS_PER_SUB, ROWS_PER_SUB)])
                plsc.subcore_barrier()


def _aggregate(rows3, cols3, hts, e_all, zsrc):
    mesh = plsc.VectorSubcoreMesh(core_axis_name="c", subcore_axis_name="s")
    f = pl.kernel(
        _scb_body,
        out_type=jax.ShapeDtypeStruct((T, NC, N_PAD, D), jnp.float32),
        mesh=mesh,
        scratch_types=[
            pltpu.VMEM_SHARED((N_PAD, D), jnp.float32),
            pltpu.VMEM((KBMAX, CHB), jnp.int32),
            pltpu.VMEM((CHB,), jnp.int32),
            pltpu.VMEM((CHB,), jnp.int32),
            pltpu.VMEM((CHB, D), jnp.float32),
            pltpu.VMEM((CHB, D), jnp.float32),
            pltpu.VMEM((CHB, D), jnp.float32),
            pltpu.VMEM((CHB, D), jnp.float32),
            pltpu.SemaphoreType.DMA,
            pltpu.SemaphoreType.DMA,
            pltpu.SemaphoreType.DMA,
            pltpu.SemaphoreType.DMA,
            pltpu.SemaphoreType.DMA,
            pltpu.SemaphoreType.DMA,
        ],
    )
    return f(rows3, cols3, hts[0], hts[1], hts[2], hts[3], e_all, zsrc)


def _tc3_body(h_ref, parts_ref, wnn, bnn, wq, bq, wk, bk, wv, bv,
              wo, bo, gp, out_ref):
    bf = jnp.bfloat16
    dot = functools.partial(jnp.dot, preferred_element_type=jnp.float32)
    hids = []
    for t in range(T):
        x = h_ref[:, t, :]
        y = x + parts_ref[t, 0] + parts_ref[t, 1]
        for L in range(4):
            z = dot(y.astype(bf), wnn[t, L]) + bnn[t, L]
            if L < 3:
                z = _silu(z)
            y = z + y
        hids.append(_silu(y) + x)
    qs = [dot(hids[t].astype(bf), wq[...]) + bq[...] for t in range(T)]
    ks = [dot(hids[t].astype(bf), wk[...]) + bk[...] for t in range(T)]
    vs = [dot(hids[t].astype(bf), wv[...]) + bv[...] for t in range(T)]
    g = gp[...]
    inv_sqrt_hd = 0.5
    for t in range(T):
        scores = [(dot(qs[t] * ks[s_], g)) * inv_sqrt_hd for s_ in range(T)]
        mx = scores[0]
        for s_ in range(1, T):
            mx = jnp.maximum(mx, scores[s_])
        es = [jnp.exp(sc - mx) for sc in scores]
        den = es[0]
        for s_ in range(1, T):
            den = den + es[s_]
        o = (es[0] / den) * vs[0]
        for s_ in range(1, T):
            o = o + (es[s_] / den) * vs[s_]
        out_ref[:, t, :] = dot(o.astype(bf), wo[...]) + bo[...] + hids[t]


def _node_stage(h, parts, wd):
    bn = 1000
    const = lambda *shape: pl.BlockSpec(shape, lambda i: tuple(0 for _ in shape))
    return pl.pallas_call(
        _tc3_body,
        grid=(N // bn,),
        in_specs=[
            pl.BlockSpec((bn, T, D), lambda i: (i, 0, 0)),
            pl.BlockSpec((T, NC, bn, D), lambda i: (0, 0, i, 0)),
            const(T, 4, D, D),
            const(T, 4, 1, D),
            const(D, D), const(1, D),
            const(D, D), const(1, D),
            const(D, D), const(1, D),
            const(D, D), const(1, D),
            const(D, D),
        ],
        out_specs=pl.BlockSpec((bn, T, D), lambda i: (i, 0, 0)),
        out_shape=jax.ShapeDtypeStruct((N, T, D), jnp.float32),
    )(h, parts, *wd)


def kernel(h, edge_index, edge_attr, edge_mask, params):
    h = h.astype(jnp.float32)
    rows = edge_index[0].astype(jnp.int32)
    cols = edge_index[1].astype(jnp.int32)
    pad = E_PAD - E
    rows_p = jnp.pad(rows, (0, pad))
    cols_p = jnp.pad(cols, (0, pad))
    bf = jnp.bfloat16
    ea_p = jnp.pad(edge_attr.astype(bf), ((0, pad), (0, 0)))
    mf_p = jnp.pad(edge_mask.astype(jnp.float32), ((0, 0), (0, pad)))

    (w1, b1), (w2, b2), (w3, b3) = params["edge_mlp"]
    w1a, w1b, w1c = w1[:D], w1[D:2 * D], w1[2 * D:]
    b1_eff = b1 + params["conv_b"] * (w1a.sum(0) + w1b.sum(0))
    wec = jnp.concatenate([g["We"] for g in params["gine"]], axis=1)
    bec = jnp.concatenate([g["be"] for g in params["gine"]], axis=0)
    rm = jnp.repeat(jnp.eye(T, dtype=jnp.float32), D, axis=1)
    wd2 = (w1a.astype(bf), w1b.astype(bf), w1c.astype(bf),
           b1_eff.reshape(1, -1), w2.astype(bf), b2.reshape(1, -1),
           w3.astype(bf), b3.reshape(1, -1), wec.astype(bf),
           bec.reshape(1, -1), rm)

    wnn = jnp.stack([jnp.stack([W for (W, _) in g["nn"]]) for g in params["gine"]])
    bnn = jnp.stack([jnp.stack([b.reshape(1, D) for (_, b) in g["nn"]])
                     for g in params["gine"]])
    at = params["attn"]
    gp = jnp.asarray(np.kron(np.eye(HEADS, dtype=np.float32),
                             np.ones((D // HEADS, D // HEADS), np.float32)))
    wd3 = (wnn.astype(bf), bnn, at["Wq"].astype(bf), at["bq"].reshape(1, D),
           at["Wk"].astype(bf), at["bk"].reshape(1, D), at["Wv"].astype(bf),
           at["bv"].reshape(1, D), at["Wo"].astype(bf),
           at["bo"].reshape(1, D), gp)

    rows3 = rows_p.reshape(NW, CHUNKS, CH)
    cols3 = cols_p.reshape(NW, CHUNKS, CH)
    rows2b = rows_p.reshape(E_PAD // CHB, CHB)
    cols2b = cols_p.reshape(E_PAD // CHB, 1, CHB)
    node_emb = _node_emb(h, params["conv_w"])
    gr, gc = _gather_pair(node_emb, rows3, cols3)
    mij_p, e_all = _edge_mlp(gr, gc, ea_p, mf_p, wd2)
    hts = [h[:, t, :] for t in range(T)]
    zsrc = jnp.zeros((ROWS_PER_SUB, D), jnp.float32)
    parts = _aggregate(rows2b, cols2b, hts, e_all, zsrc)
    node_vec = _node_stage(h, parts, wd3)
    return node_vec, mij_p[:E]


# speedup vs baseline: 1.1027x; 1.1027x over previous
import functools

import jax
import jax.numpy as jnp
import numpy as np
from jax import lax
from jax.experimental import pallas as pl
from jax.experimental.pallas import tpu as pltpu
from jax.experimental.pallas import tpu_sc as plsc

N = 10000
E = 160000
D = 128
ED = 16
T = 4
HEADS = 32

NC = 2
NS = 16
NW = NC * NS
CH = 128
CHUNKS = 40
EPW = CH * CHUNKS
CHB = 64
CHUNKS_B = EPW // CHB
KB0 = 104
KB1 = 56
KBMAX = max(KB0, KB1)
E_PAD = NW * EPW
N_PAD = 10240
ROWS_PER_SUB = N_PAD // NS
NEG = -1.0e30


def _silu(x):
    return x * jax.nn.sigmoid(x)


def _tc1_body(h_ref, w_ref, o_ref):
    acc = h_ref[:, 0, :] * w_ref[0]
    for c in range(1, T):
        acc = acc + h_ref[:, c, :] * w_ref[c]
    o_ref[...] = acc


def _node_emb(h, conv_w):
    bn = 1000
    return pl.pallas_call(
        _tc1_body,
        grid=(N // bn,),
        in_specs=[
            pl.BlockSpec((bn, T, D), lambda i: (i, 0, 0)),
            pl.BlockSpec(memory_space=pltpu.SMEM),
        ],
        out_specs=pl.BlockSpec((bn, D), lambda i: (i, 0)),
        out_shape=jax.ShapeDtypeStruct((N, D), jnp.float32),
    )(h, conv_w)


def _sca_body(ne_hbm, rows3_hbm, cols3_hbm, gr_hbm, gc_hbm,
              ridx_v, cidx_v, br0, bc0, br1, bc1, sg0, sg1, ss0, ss1):
    c = lax.axis_index("c")
    s = lax.axis_index("s")
    wid = c * NS + s
    base0 = wid * EPW
    pltpu.sync_copy(rows3_hbm.at[wid], ridx_v)
    pltpu.sync_copy(cols3_hbm.at[wid], cidx_v)

    def issue_g(j, br, bc, sg):
        pltpu.async_copy(ne_hbm.at[ridx_v.at[j]], br, sg)
        pltpu.async_copy(ne_hbm.at[cidx_v.at[j]], bc, sg)

    def wait_g(br, bc, sg):
        pltpu.make_async_copy(ne_hbm.at[pl.ds(0, CH)], br, sg).wait()
        pltpu.make_async_copy(ne_hbm.at[pl.ds(0, CH)], bc, sg).wait()

    def wait_s(br, bc, ss):
        pltpu.make_async_copy(br, gr_hbm.at[pl.ds(0, CH)], ss).wait()
        pltpu.make_async_copy(bc, gc_hbm.at[pl.ds(0, CH)], ss).wait()

    issue_g(0, br0, bc0, sg0)
    issue_g(1, br1, bc1, sg1)

    @pl.loop(0, CHUNKS, step=2)
    def _(j0):
        base = base0 + j0 * CH
        wait_g(br0, bc0, sg0)
        pltpu.async_copy(br0, gr_hbm.at[pl.ds(base, CH)], ss0)
        pltpu.async_copy(bc0, gc_hbm.at[pl.ds(base, CH)], ss0)
        wait_g(br1, bc1, sg1)
        pltpu.async_copy(br1, gr_hbm.at[pl.ds(base + CH, CH)], ss1)
        pltpu.async_copy(bc1, gc_hbm.at[pl.ds(base + CH, CH)], ss1)
        wait_s(br0, bc0, ss0)

        @pl.when(j0 + 2 < CHUNKS)
        def _():
            issue_g(j0 + 2, br0, bc0, sg0)

        wait_s(br1, bc1, ss1)

        @pl.when(j0 + 3 < CHUNKS)
        def _():
            issue_g(j0 + 3, br1, bc1, sg1)


def _gather_pair(node_emb, rows3, cols3):
    mesh = plsc.VectorSubcoreMesh(core_axis_name="c", subcore_axis_name="s")
    f = pl.kernel(
        _sca_body,
        out_type=[
            jax.ShapeDtypeStruct((E_PAD, D), jnp.float32),
            jax.ShapeDtypeStruct((E_PAD, D), jnp.float32),
        ],
        mesh=mesh,
        scratch_types=[
            pltpu.VMEM((CHUNKS, CH), jnp.int32),
            pltpu.VMEM((CHUNKS, CH), jnp.int32),
            pltpu.VMEM((CH, D), jnp.float32),
            pltpu.VMEM((CH, D), jnp.float32),
            pltpu.VMEM((CH, D), jnp.float32),
            pltpu.VMEM((CH, D), jnp.float32),
            pltpu.SemaphoreType.DMA,
            pltpu.SemaphoreType.DMA,
            pltpu.SemaphoreType.DMA,
            pltpu.SemaphoreType.DMA,
        ],
    )
    return f(node_emb, rows3, cols3)


def _tc2_body(gr, gc, ea, mf, w1a, w1b, w1c, b1, w2, b2, w3, b3, wec, bec,
              rm, mij_ref, e_ref):
    bf = jnp.bfloat16
    f32 = jnp.float32
    dot = functools.partial(jnp.dot, preferred_element_type=f32)
    y = (dot(gr[...].astype(bf), w1a[...]) + dot(gc[...].astype(bf), w1b[...])
         + dot(ea[...], w1c[...]) + b1[...])
    y = _silu(y)
    y = dot(y.astype(bf), w2[...]) + b2[...]
    y = _silu(y)
    m = dot(y.astype(bf), w3[...]) + b3[...]
    mij_ref[...] = m
    e4 = dot(m.astype(bf), wec[...]) + bec[...]
    mrep = lax.dot_general(mf[...], rm[...], (((0,), (0,)), ((), ())),
                           preferred_element_type=f32)
    e4 = jnp.where(mrep > 0.5, e4, NEG)
    for t in range(T):
        e_ref[t, :, :] = e4[:, t * D:(t + 1) * D]


def _edge_mlp(gr, gc, ea_p, mf_p, wd):
    be = 1024
    grid = (E_PAD // be,)
    const = lambda *shape: pl.BlockSpec(shape, lambda i: tuple(0 for _ in shape))
    return pl.pallas_call(
        _tc2_body,
        grid=grid,
        in_specs=[
            pl.BlockSpec((be, D), lambda i: (i, 0)),
            pl.BlockSpec((be, D), lambda i: (i, 0)),
            pl.BlockSpec((be, ED), lambda i: (i, 0)),
            pl.BlockSpec((T, be), lambda i: (0, i)),
            const(D, 3 * D),
            const(D, 3 * D),
            const(ED, 3 * D),
            const(1, 3 * D),
            const(3 * D, 2 * D),
            const(1, 2 * D),
            const(2 * D, D),
            const(1, D),
            const(D, T * D),
            const(1, T * D),
            const(T, T * D),
        ],
        out_specs=[
            pl.BlockSpec((be, D), lambda i: (i, 0)),
            pl.BlockSpec((T, be, D), lambda i: (0, i, 0)),
        ],
        out_shape=[
            jax.ShapeDtypeStruct((E_PAD, D), jnp.float32),
            jax.ShapeDtypeStruct((T, E_PAD, D), jnp.float32),
        ],
    )(gr, gc, ea_p, mf_p, *wd)


def _scb_body(rows2_hbm, cols2_hbm, h0, h1, h2, h3, e_hbm, z_hbm, out_hbm,
              accum, ridx_v, cidx0, cidx1, eb0, xb0, eb1, xb1,
              se0, se1, sg0, sg1, ss0, ss1):
    c = lax.axis_index("c")
    s = lax.axis_index("s")
    hts = (h0, h1, h2, h3)

    for core in range(NC):
        K = (KB0, KB1)[core]
        cbase = s * KB0 if core == 0 else NS * KB0 + s * KB1

        @pl.when(c == core)
        def _(K=K, cbase=cbase):
            pltpu.sync_copy(rows2_hbm.at[pl.ds(cbase, K)],
                            ridx_v.at[pl.ds(0, K)])
            for t in range(T):
                ht = hts[t]

                def issue_pair(j, eb, xb, cidx, se, sg):
                    ch = cbase + j
                    pltpu.async_copy(e_hbm.at[t, pl.ds(ch * CHB, CHB)], eb, se)
                    pltpu.async_copy(cols2_hbm.at[ch, 0], cidx, se)
                    pltpu.async_copy(ht.at[ridx_v.at[j]], xb, sg)

                def half(j, eb, xb, cidx, se, sg, ss):
                    pltpu.make_async_copy(e_hbm.at[0, pl.ds(0, CHB)],
                                          eb, se).wait()
                    pltpu.make_async_copy(cols2_hbm.at[0, 0], cidx, se).wait()
                    pltpu.make_async_copy(ht.at[pl.ds(0, CHB)], xb, sg).wait()

                    @plsc.parallel_loop(0, CHB, unroll=2)
                    def _(r):
                        for q in range(D // 16):
                            sl = pl.ds(q * 16, 16)
                            eb[r, sl] = jnp.maximum(eb[r, sl] + xb[r, sl], 0.0)

                    pltpu.async_copy(eb, accum.at[cidx], ss, add=True)

                pltpu.sync_copy(z_hbm.at[pl.ds(0, ROWS_PER_SUB)],
                                accum.at[pl.ds(s * ROWS_PER_SUB, ROWS_PER_SUB)])
                plsc.subcore_barrier()

                issue_pair(0, eb0, xb0, cidx0, se0, sg0)
                issue_pair(1, eb1, xb1, cidx1, se1, sg1)

                @pl.loop(0, K, step=2)
                def _(j0):
                    half(j0, eb0, xb0, cidx0, se0, sg0, ss0)
                    half(j0 + 1, eb1, xb1, cidx1, se1, sg1, ss1)
                    pltpu.make_async_copy(eb0, accum.at[cidx0], ss0).wait()

                    @pl.when(j0 + 2 < K)
                    def _():
                        issue_pair(j0 + 2, eb0, xb0, cidx0, se0, sg0)

                    pltpu.make_async_copy(eb1, accum.at[cidx1], ss1).wait()

                    @pl.when(j0 + 3 < K)
                    def _():
                        issue_pair(j0 + 3, eb1, xb1, cidx1, se1, sg1)

                plsc.subcore_barrier()
                pltpu.sync_copy(
                    accum.at[pl.ds(s * ROWS_PER_SUB, ROWS_PER_SUB)],
                    out_hbm.at[t, c, pl.ds(s * ROWS_PER_SUB, ROWS_PER_SUB)])
                plsc.subcore_barrier()


def _aggregate(rows3, cols3, hts, e_all, zsrc):
    mesh = plsc.VectorSubcoreMesh(core_axis_name="c", subcore_axis_name="s")
    f = pl.kernel(
        _scb_body,
        out_type=jax.ShapeDtypeStruct((T, NC, N_PAD, D), jnp.float32),
        mesh=mesh,
        scratch_types=[
            pltpu.VMEM_SHARED((N_PAD, D), jnp.float32),
            pltpu.VMEM((KBMAX, CHB), jnp.int32),
            pltpu.VMEM((CHB,), jnp.int32),
            pltpu.VMEM((CHB,), jnp.int32),
            pltpu.VMEM((CHB, D), jnp.float32),
            pltpu.VMEM((CHB, D), jnp.float32),
            pltpu.VMEM((CHB, D), jnp.float32),
            pltpu.VMEM((CHB, D), jnp.float32),
            pltpu.SemaphoreType.DMA,
            pltpu.SemaphoreType.DMA,
            pltpu.SemaphoreType.DMA,
            pltpu.SemaphoreType.DMA,
            pltpu.SemaphoreType.DMA,
            pltpu.SemaphoreType.DMA,
        ],
    )
    return f(rows3, cols3, hts[0], hts[1], hts[2], hts[3], e_all, zsrc)


def _tc3_body(h_ref, parts_ref, wnn, bnn, wq, bq, wk, bk, wv, bv,
              wo, bo, gp, out_ref):
    bf = jnp.bfloat16
    dot = functools.partial(jnp.dot, preferred_element_type=jnp.float32)
    hids = []
    for t in range(T):
        x = h_ref[:, t, :]
        y = x + parts_ref[t, 0] + parts_ref[t, 1]
        for L in range(4):
            z = dot(y.astype(bf), wnn[t, L]) + bnn[t, L]
            if L < 3:
                z = _silu(z)
            y = z + y
        hids.append(_silu(y) + x)
    qs = [dot(hids[t].astype(bf), wq[...]) + bq[...] for t in range(T)]
    ks = [dot(hids[t].astype(bf), wk[...]) + bk[...] for t in range(T)]
    vs = [dot(hids[t].astype(bf), wv[...]) + bv[...] for t in range(T)]
    g = gp[...]
    inv_sqrt_hd = 0.5
    for t in range(T):
        scores = [(dot(qs[t] * ks[s_], g)) * inv_sqrt_hd for s_ in range(T)]
        mx = scores[0]
        for s_ in range(1, T):
            mx = jnp.maximum(mx, scores[s_])
        es = [jnp.exp(sc - mx) for sc in scores]
        den = es[0]
        for s_ in range(1, T):
            den = den + es[s_]
        o = (es[0] / den) * vs[0]
        for s_ in range(1, T):
            o = o + (es[s_] / den) * vs[s_]
        out_ref[:, t, :] = dot(o.astype(bf), wo[...]) + bo[...] + hids[t]


def _node_stage(h, parts, wd):
    bn = 1000
    const = lambda *shape: pl.BlockSpec(shape, lambda i: tuple(0 for _ in shape))
    return pl.pallas_call(
        _tc3_body,
        grid=(N // bn,),
        in_specs=[
            pl.BlockSpec((bn, T, D), lambda i: (i, 0, 0)),
            pl.BlockSpec((T, NC, bn, D), lambda i: (0, 0, i, 0)),
            const(T, 4, D, D),
            const(T, 4, 1, D),
            const(D, D), const(1, D),
            const(D, D), const(1, D),
            const(D, D), const(1, D),
            const(D, D), const(1, D),
            const(D, D),
        ],
        out_specs=pl.BlockSpec((bn, T, D), lambda i: (i, 0, 0)),
        out_shape=jax.ShapeDtypeStruct((N, T, D), jnp.float32),
    )(h, parts, *wd)


def kernel(h, edge_index, edge_attr, edge_mask, params):
    h = h.astype(jnp.float32)
    rows = edge_index[0].astype(jnp.int32)
    cols = edge_index[1].astype(jnp.int32)
    pad = E_PAD - E
    rows_p = jnp.pad(rows, (0, pad))
    cols_p = jnp.pad(cols, (0, pad))
    bf = jnp.bfloat16
    ea_p = jnp.pad(edge_attr.astype(bf), ((0, pad), (0, 0)))
    mf_p = jnp.pad(edge_mask.astype(jnp.float32), ((0, 0), (0, pad)))

    (w1, b1), (w2, b2), (w3, b3) = params["edge_mlp"]
    w1a, w1b, w1c = w1[:D], w1[D:2 * D], w1[2 * D:]
    b1_eff = b1 + params["conv_b"] * (w1a.sum(0) + w1b.sum(0))
    wec = jnp.concatenate([g["We"] for g in params["gine"]], axis=1)
    bec = jnp.concatenate([g["be"] for g in params["gine"]], axis=0)
    rm = jnp.repeat(jnp.eye(T, dtype=jnp.float32), D, axis=1)
    wd2 = (w1a.astype(bf), w1b.astype(bf), w1c.astype(bf),
           b1_eff.reshape(1, -1), w2.astype(bf), b2.reshape(1, -1),
           w3.astype(bf), b3.reshape(1, -1), wec.astype(bf),
           bec.reshape(1, -1), rm)

    wnn = jnp.stack([jnp.stack([W for (W, _) in g["nn"]]) for g in params["gine"]])
    bnn = jnp.stack([jnp.stack([b.reshape(1, D) for (_, b) in g["nn"]])
                     for g in params["gine"]])
    at = params["attn"]
    gp = jnp.asarray(np.kron(np.eye(HEADS, dtype=np.float32),
                             np.ones((D // HEADS, D // HEADS), np.float32)))
    wd3 = (wnn.astype(bf), bnn, at["Wq"].astype(bf), at["bq"].reshape(1, D),
           at["Wk"].astype(bf), at["bk"].reshape(1, D), at["Wv"].astype(bf),
           at["bv"].reshape(1, D), at["Wo"].astype(bf),
           at["bo"].reshape(1, D), gp)

    rows3 = rows_p.reshape(NW, CHUNKS, CH)
    cols3 = cols_p.reshape(NW, CHUNKS, CH)
    rows2b = rows_p.reshape(E_PAD // CHB, CHB)
    cols2b = cols_p.reshape(E_PAD // CHB, 1, CHB)
    node_emb = _node_emb(h, params["conv_w"])
    gr, gc = _gather_pair(node_emb, rows3, cols3)
    mij_p, e_all = _edge_mlp(gr, gc, ea_p, mf_p, wd2)
    hts = [h[:, t, :] for t in range(T)]
    zsrc = jnp.zeros((ROWS_PER_SUB, D), jnp.float32)
    parts = _aggregate(rows2b, cols2b, hts, e_all, zsrc)
    node_vec = _node_stage(h, parts, wd3)
    return node_vec, mij_p[:E]


# speedup vs baseline: 1.1125x; 1.0088x over previous
import functools

import jax
import jax.numpy as jnp
import numpy as np
from jax import lax
from jax.experimental import pallas as pl
from jax.experimental.pallas import tpu as pltpu
from jax.experimental.pallas import tpu_sc as plsc

N = 10000
E = 160000
D = 128
ED = 16
T = 4
HEADS = 32

NC = 2
NS = 16
NW = NC * NS
CH = 128
CHUNKS = 40
EPW = CH * CHUNKS
CHB = 64
CHUNKS_B = EPW // CHB
KB0 = 104
KB1 = 56
KBMAX = max(KB0, KB1)
KA0 = 64
KA1 = 16
KAMAX = max(KA0, KA1)
E_PAD = NW * EPW
N_PAD = 10240
ROWS_PER_SUB = N_PAD // NS
NEG = -1.0e30


def _silu(x):
    return x * jax.nn.sigmoid(x)


def _tc1_body(h_ref, w_ref, o_ref):
    acc = h_ref[:, 0, :] * w_ref[0]
    for c in range(1, T):
        acc = acc + h_ref[:, c, :] * w_ref[c]
    o_ref[...] = acc


def _node_emb(h, conv_w):
    bn = 1000
    return pl.pallas_call(
        _tc1_body,
        grid=(N // bn,),
        in_specs=[
            pl.BlockSpec((bn, T, D), lambda i: (i, 0, 0)),
            pl.BlockSpec(memory_space=pltpu.SMEM),
        ],
        out_specs=pl.BlockSpec((bn, D), lambda i: (i, 0)),
        out_shape=jax.ShapeDtypeStruct((N, D), jnp.float32),
    )(h, conv_w)


def _sca_body(ne_hbm, rows2_hbm, cols2_hbm, gr_hbm, gc_hbm,
              ridx_v, cidx_v, br0, bc0, br1, bc1, sg0, sg1, ss0, ss1):
    c = lax.axis_index("c")
    s = lax.axis_index("s")

    def wait_g(br, bc, sg):
        pltpu.make_async_copy(ne_hbm.at[pl.ds(0, CH)], br, sg).wait()
        pltpu.make_async_copy(ne_hbm.at[pl.ds(0, CH)], bc, sg).wait()

    def wait_s(br, bc, ss):
        pltpu.make_async_copy(br, gr_hbm.at[pl.ds(0, CH)], ss).wait()
        pltpu.make_async_copy(bc, gc_hbm.at[pl.ds(0, CH)], ss).wait()

    for core in range(NC):
        K = (KA0, KA1)[core]
        cbase = s * KA0 if core == 0 else NS * KA0 + s * KA1

        @pl.when(c == core)
        def _(K=K, cbase=cbase):
            pltpu.sync_copy(rows2_hbm.at[pl.ds(cbase, K)],
                            ridx_v.at[pl.ds(0, K)])
            pltpu.sync_copy(cols2_hbm.at[pl.ds(cbase, K)],
                            cidx_v.at[pl.ds(0, K)])

            def issue_g(j, br, bc, sg):
                pltpu.async_copy(ne_hbm.at[ridx_v.at[j]], br, sg)
                pltpu.async_copy(ne_hbm.at[cidx_v.at[j]], bc, sg)

            issue_g(0, br0, bc0, sg0)
            issue_g(1, br1, bc1, sg1)

            @pl.loop(0, K, step=2)
            def _(j0):
                base = (cbase + j0) * CH
                wait_g(br0, bc0, sg0)
                pltpu.async_copy(br0, gr_hbm.at[pl.ds(base, CH)], ss0)
                pltpu.async_copy(bc0, gc_hbm.at[pl.ds(base, CH)], ss0)
                wait_g(br1, bc1, sg1)
                pltpu.async_copy(br1, gr_hbm.at[pl.ds(base + CH, CH)], ss1)
                pltpu.async_copy(bc1, gc_hbm.at[pl.ds(base + CH, CH)], ss1)
                wait_s(br0, bc0, ss0)

                @pl.when(j0 + 2 < K)
                def _():
                    issue_g(j0 + 2, br0, bc0, sg0)

                wait_s(br1, bc1, ss1)

                @pl.when(j0 + 3 < K)
                def _():
                    issue_g(j0 + 3, br1, bc1, sg1)


def _gather_pair(node_emb, rows3, cols3):
    mesh = plsc.VectorSubcoreMesh(core_axis_name="c", subcore_axis_name="s")
    f = pl.kernel(
        _sca_body,
        out_type=[
            jax.ShapeDtypeStruct((E_PAD, D), jnp.float32),
            jax.ShapeDtypeStruct((E_PAD, D), jnp.float32),
        ],
        mesh=mesh,
        scratch_types=[
            pltpu.VMEM((KAMAX, CH), jnp.int32),
            pltpu.VMEM((KAMAX, CH), jnp.int32),
            pltpu.VMEM((CH, D), jnp.float32),
            pltpu.VMEM((CH, D), jnp.float32),
            pltpu.VMEM((CH, D), jnp.float32),
            pltpu.VMEM((CH, D), jnp.float32),
            pltpu.SemaphoreType.DMA,
            pltpu.SemaphoreType.DMA,
            pltpu.SemaphoreType.DMA,
            pltpu.SemaphoreType.DMA,
        ],
    )
    return f(node_emb, rows3, cols3)


def _tc2_body(gr, gc, ea, mf, w1a, w1b, w1c, b1, w2, b2, w3, b3, wec, bec,
              rm, mij_ref, e_ref):
    bf = jnp.bfloat16
    f32 = jnp.float32
    dot = functools.partial(jnp.dot, preferred_element_type=f32)
    y = (dot(gr[...].astype(bf), w1a[...]) + dot(gc[...].astype(bf), w1b[...])
         + dot(ea[...], w1c[...]) + b1[...])
    y = _silu(y)
    y = dot(y.astype(bf), w2[...]) + b2[...]
    y = _silu(y)
    m = dot(y.astype(bf), w3[...]) + b3[...]
    mij_ref[...] = m
    e4 = dot(m.astype(bf), wec[...]) + bec[...]
    mrep = lax.dot_general(mf[...], rm[...], (((0,), (0,)), ((), ())),
                           preferred_element_type=f32)
    e4 = jnp.where(mrep > 0.5, e4, NEG)
    for t in range(T):
        e_ref[t, :, :] = e4[:, t * D:(t + 1) * D]


def _edge_mlp(gr, gc, ea_p, mf_p, wd):
    be = 1024
    grid = (E_PAD // be,)
    const = lambda *shape: pl.BlockSpec(shape, lambda i: tuple(0 for _ in shape))
    return pl.pallas_call(
        _tc2_body,
        grid=grid,
        in_specs=[
            pl.BlockSpec((be, D), lambda i: (i, 0)),
            pl.BlockSpec((be, D), lambda i: (i, 0)),
            pl.BlockSpec((be, ED), lambda i: (i, 0)),
            pl.BlockSpec((T, be), lambda i: (0, i)),
            const(D, 3 * D),
            const(D, 3 * D),
            const(ED, 3 * D),
            const(1, 3 * D),
            const(3 * D, 2 * D),
            const(1, 2 * D),
            const(2 * D, D),
            const(1, D),
            const(D, T * D),
            const(1, T * D),
            const(T, T * D),
        ],
        out_specs=[
            pl.BlockSpec((be, D), lambda i: (i, 0)),
            pl.BlockSpec((T, be, D), lambda i: (0, i, 0)),
        ],
        out_shape=[
            jax.ShapeDtypeStruct((E_PAD, D), jnp.float32),
            jax.ShapeDtypeStruct((T, E_PAD, D), jnp.float32),
        ],
    )(gr, gc, ea_p, mf_p, *wd)


def _scb_body(rows2_hbm, cols2_hbm, h0, h1, h2, h3, e_hbm, z_hbm, out_hbm,
              accum, ridx_v, cidx0, cidx1, eb0, xb0, eb1, xb1,
              se0, se1, sg0, sg1, ss0, ss1):
    c = lax.axis_index("c")
    s = lax.axis_index("s")
    hts = (h0, h1, h2, h3)

    for core in range(NC):
        K = (KB0, KB1)[core]
        cbase = s * KB0 if core == 0 else NS * KB0 + s * KB1

        @pl.when(c == core)
        def _(K=K, cbase=cbase):
            pltpu.sync_copy(rows2_hbm.at[pl.ds(cbase, K)],
                            ridx_v.at[pl.ds(0, K)])
            for t in range(T):
                ht = hts[t]

                def issue_pair(j, eb, xb, cidx, se, sg):
                    ch = cbase + j
                    pltpu.async_copy(e_hbm.at[t, pl.ds(ch * CHB, CHB)], eb, se)
                    pltpu.async_copy(cols2_hbm.at[ch, 0], cidx, se)
                    pltpu.async_copy(ht.at[ridx_v.at[j]], xb, sg)

                def half(j, eb, xb, cidx, se, sg, ss):
                    pltpu.make_async_copy(e_hbm.at[0, pl.ds(0, CHB)],
                                          eb, se).wait()
                    pltpu.make_async_copy(cols2_hbm.at[0, 0], cidx, se).wait()
                    pltpu.make_async_copy(ht.at[pl.ds(0, CHB)], xb, sg).wait()

                    @plsc.parallel_loop(0, CHB, unroll=2)
                    def _(r):
                        for q in range(D // 16):
                            sl = pl.ds(q * 16, 16)
                            eb[r, sl] = jnp.maximum(eb[r, sl] + xb[r, sl], 0.0)

                    pltpu.async_copy(eb, accum.at[cidx], ss, add=True)

                pltpu.sync_copy(z_hbm.at[pl.ds(0, ROWS_PER_SUB)],
                                accum.at[pl.ds(s * ROWS_PER_SUB, ROWS_PER_SUB)])
                plsc.subcore_barrier()

                issue_pair(0, eb0, xb0, cidx0, se0, sg0)
                issue_pair(1, eb1, xb1, cidx1, se1, sg1)

                @pl.loop(0, K, step=2)
                def _(j0):
                    half(j0, eb0, xb0, cidx0, se0, sg0, ss0)
                    half(j0 + 1, eb1, xb1, cidx1, se1, sg1, ss1)
                    pltpu.make_async_copy(eb0, accum.at[cidx0], ss0).wait()

                    @pl.when(j0 + 2 < K)
                    def _():
                        issue_pair(j0 + 2, eb0, xb0, cidx0, se0, sg0)

                    pltpu.make_async_copy(eb1, accum.at[cidx1], ss1).wait()

                    @pl.when(j0 + 3 < K)
                    def _():
                        issue_pair(j0 + 3, eb1, xb1, cidx1, se1, sg1)

                plsc.subcore_barrier()
                pltpu.sync_copy(
                    accum.at[pl.ds(s * ROWS_PER_SUB, ROWS_PER_SUB)],
                    out_hbm.at[t, c, pl.ds(s * ROWS_PER_SUB, ROWS_PER_SUB)])
                plsc.subcore_barrier()


def _aggregate(rows3, cols3, hts, e_all, zsrc):
    mesh = plsc.VectorSubcoreMesh(core_axis_name="c", subcore_axis_name="s")
    f = pl.kernel(
        _scb_body,
        out_type=jax.ShapeDtypeStruct((T, NC, N_PAD, D), jnp.float32),
        mesh=mesh,
        scratch_types=[
            pltpu.VMEM_SHARED((N_PAD, D), jnp.float32),
            pltpu.VMEM((KBMAX, CHB), jnp.int32),
            pltpu.VMEM((CHB,), jnp.int32),
            pltpu.VMEM((CHB,), jnp.int32),
            pltpu.VMEM((CHB, D), jnp.float32),
            pltpu.VMEM((CHB, D), jnp.float32),
            pltpu.VMEM((CHB, D), jnp.float32),
            pltpu.VMEM((CHB, D), jnp.float32),
            pltpu.SemaphoreType.DMA,
            pltpu.SemaphoreType.DMA,
            pltpu.SemaphoreType.DMA,
            pltpu.SemaphoreType.DMA,
            pltpu.SemaphoreType.DMA,
            pltpu.SemaphoreType.DMA,
        ],
    )
    return f(rows3, cols3, hts[0], hts[1], hts[2], hts[3], e_all, zsrc)


def _tc3_body(h_ref, parts_ref, wnn, bnn, wq, bq, wk, bk, wv, bv,
              wo, bo, gp, out_ref):
    bf = jnp.bfloat16
    dot = functools.partial(jnp.dot, preferred_element_type=jnp.float32)
    hids = []
    for t in range(T):
        x = h_ref[:, t, :]
        y = x + parts_ref[t, 0] + parts_ref[t, 1]
        for L in range(4):
            z = dot(y.astype(bf), wnn[t, L]) + bnn[t, L]
            if L < 3:
                z = _silu(z)
            y = z + y
        hids.append(_silu(y) + x)
    qs = [dot(hids[t].astype(bf), wq[...]) + bq[...] for t in range(T)]
    ks = [dot(hids[t].astype(bf), wk[...]) + bk[...] for t in range(T)]
    vs = [dot(hids[t].astype(bf), wv[...]) + bv[...] for t in range(T)]
    g = gp[...]
    inv_sqrt_hd = 0.5
    for t in range(T):
        scores = [(dot(qs[t] * ks[s_], g)) * inv_sqrt_hd for s_ in range(T)]
        mx = scores[0]
        for s_ in range(1, T):
            mx = jnp.maximum(mx, scores[s_])
        es = [jnp.exp(sc - mx) for sc in scores]
        den = es[0]
        for s_ in range(1, T):
            den = den + es[s_]
        o = (es[0] / den) * vs[0]
        for s_ in range(1, T):
            o = o + (es[s_] / den) * vs[s_]
        out_ref[:, t, :] = dot(o.astype(bf), wo[...]) + bo[...] + hids[t]


def _node_stage(h, parts, wd):
    bn = 1000
    const = lambda *shape: pl.BlockSpec(shape, lambda i: tuple(0 for _ in shape))
    return pl.pallas_call(
        _tc3_body,
        grid=(N // bn,),
        in_specs=[
            pl.BlockSpec((bn, T, D), lambda i: (i, 0, 0)),
            pl.BlockSpec((T, NC, bn, D), lambda i: (0, 0, i, 0)),
            const(T, 4, D, D),
            const(T, 4, 1, D),
            const(D, D), const(1, D),
            const(D, D), const(1, D),
            const(D, D), const(1, D),
            const(D, D), const(1, D),
            const(D, D),
        ],
        out_specs=pl.BlockSpec((bn, T, D), lambda i: (i, 0, 0)),
        out_shape=jax.ShapeDtypeStruct((N, T, D), jnp.float32),
    )(h, parts, *wd)


def kernel(h, edge_index, edge_attr, edge_mask, params):
    h = h.astype(jnp.float32)
    rows = edge_index[0].astype(jnp.int32)
    cols = edge_index[1].astype(jnp.int32)
    pad = E_PAD - E
    rows_p = jnp.pad(rows, (0, pad))
    cols_p = jnp.pad(cols, (0, pad))
    bf = jnp.bfloat16
    ea_p = jnp.pad(edge_attr.astype(bf), ((0, pad), (0, 0)))
    mf_p = jnp.pad(edge_mask.astype(jnp.float32), ((0, 0), (0, pad)))

    (w1, b1), (w2, b2), (w3, b3) = params["edge_mlp"]
    w1a, w1b, w1c = w1[:D], w1[D:2 * D], w1[2 * D:]
    b1_eff = b1 + params["conv_b"] * (w1a.sum(0) + w1b.sum(0))
    wec = jnp.concatenate([g["We"] for g in params["gine"]], axis=1)
    bec = jnp.concatenate([g["be"] for g in params["gine"]], axis=0)
    rm = jnp.repeat(jnp.eye(T, dtype=jnp.float32), D, axis=1)
    wd2 = (w1a.astype(bf), w1b.astype(bf), w1c.astype(bf),
           b1_eff.reshape(1, -1), w2.astype(bf), b2.reshape(1, -1),
           w3.astype(bf), b3.reshape(1, -1), wec.astype(bf),
           bec.reshape(1, -1), rm)

    wnn = jnp.stack([jnp.stack([W for (W, _) in g["nn"]]) for g in params["gine"]])
    bnn = jnp.stack([jnp.stack([b.reshape(1, D) for (_, b) in g["nn"]])
                     for g in params["gine"]])
    at = params["attn"]
    gp = jnp.asarray(np.kron(np.eye(HEADS, dtype=np.float32),
                             np.ones((D // HEADS, D // HEADS), np.float32)))
    wd3 = (wnn.astype(bf), bnn, at["Wq"].astype(bf), at["bq"].reshape(1, D),
           at["Wk"].astype(bf), at["bk"].reshape(1, D), at["Wv"].astype(bf),
           at["bv"].reshape(1, D), at["Wo"].astype(bf),
           at["bo"].reshape(1, D), gp)

    rows2 = rows_p.reshape(E_PAD // CH, CH)
    cols2 = cols_p.reshape(E_PAD // CH, CH)
    rows2b = rows_p.reshape(E_PAD // CHB, CHB)
    cols2b = cols_p.reshape(E_PAD // CHB, 1, CHB)
    node_emb = _node_emb(h, params["conv_w"])
    gr, gc = _gather_pair(node_emb, rows2, cols2)
    mij_p, e_all = _edge_mlp(gr, gc, ea_p, mf_p, wd2)
    hts = [h[:, t, :] for t in range(T)]
    zsrc = jnp.zeros((ROWS_PER_SUB, D), jnp.float32)
    parts = _aggregate(rows2b, cols2b, hts, e_all, zsrc)
    node_vec = _node_stage(h, parts, wd3)
    return node_vec, mij_p[:E]
